# Initial kernel scaffold; baseline (speedup 1.0000x reference)
import jax
import jax.numpy as jnp
from jax.experimental import pallas as pl


def kernel(feat, edge_index, edge_weight, W):
    raise NotImplementedError("write your pallas kernel here")


# speedup vs baseline: 3.6433x; 3.6433x over previous
import functools

import jax
import jax.numpy as jnp
from jax import lax
from jax.experimental import pallas as pl
from jax.experimental.pallas import tpu as pltpu
from jax.experimental.pallas import tpu_sc as plsc

N_NODES = 10000
N_EDGES = 320000
IN_FEAT = 128
OUT_FEAT = 128

NUM_CORES = 2
NUM_SUBCORES = 16
NW = NUM_CORES * NUM_SUBCORES
E_PER_TILE = N_EDGES // NW
CHUNK = 80
N_CHUNKS = E_PER_TILE // CHUNK
ACC_ROWS = 10240
ROWS_PER_TILE = ACC_ROWS // NUM_SUBCORES
ZROWS = 128

MM_BLK = 1000


def _matmul_body(x_ref, w_ref, o_ref):
    o_ref[...] = jnp.dot(x_ref[...], w_ref[...],
                         preferred_element_type=jnp.float32)


def _add_body(a_ref, b_ref, o_ref):
    o_ref[...] = a_ref[...] + b_ref[...]


def _sc_body(h_hbm, src_hbm, dst_hbm, ew_hbm, out_hbm,
             acc, src_v, dst_v, ew_v, rows_v, zbuf_v, sem):
    cid = lax.axis_index("c")
    sid = lax.axis_index("s")
    wid = cid * NUM_SUBCORES + sid

    zeros16 = jnp.zeros((16,), jnp.float32)

    def _zrow(r, carry):
        for j in range(8):
            zbuf_v[r, pl.ds(j * 16, 16)] = zeros16
        return carry

    lax.fori_loop(0, ZROWS, _zrow, 0)
    row0 = sid * ROWS_PER_TILE
    for t in range(ROWS_PER_TILE // ZROWS):
        pltpu.sync_copy(zbuf_v, acc.at[pl.ds(row0 + t * ZROWS, ZROWS)])
    plsc.subcore_barrier()

    ebase = wid * E_PER_TILE

    def _chunk(k, carry):
        off = ebase + k * CHUNK
        pltpu.sync_copy(src_hbm.at[pl.ds(off, CHUNK)], src_v)
        pltpu.sync_copy(dst_hbm.at[pl.ds(off, CHUNK)], dst_v)
        pltpu.sync_copy(ew_hbm.at[pl.ds(off, CHUNK)], ew_v)
        pltpu.async_copy(h_hbm.at[src_v], rows_v, sem).wait()

        def _scale(e, c2):
            wvec = plsc.load_gather(ew_v, [jnp.full((16,), e, jnp.int32)])
            for j in range(8):
                rows_v[e, pl.ds(j * 16, 16)] = (
                    rows_v[e, pl.ds(j * 16, 16)] * wvec)
            return c2

        lax.fori_loop(0, CHUNK, _scale, 0)
        pltpu.sync_copy(rows_v, acc.at[dst_v], add=True)
        return carry

    lax.fori_loop(0, N_CHUNKS, _chunk, 0)
    plsc.subcore_barrier()

    for t in range(ROWS_PER_TILE // ZROWS):
        r = row0 + t * ZROWS
        pltpu.sync_copy(acc.at[pl.ds(r, ZROWS)], zbuf_v)
        pltpu.sync_copy(zbuf_v, out_hbm.at[cid, pl.ds(r, ZROWS)])


_sc_scatter = functools.partial(
    pl.kernel,
    mesh=plsc.VectorSubcoreMesh(core_axis_name="c", subcore_axis_name="s"),
    compiler_params=pltpu.CompilerParams(needs_layout_passes=False),
    out_type=jax.ShapeDtypeStruct((NUM_CORES, ACC_ROWS, OUT_FEAT), jnp.float32),
    scratch_types=[
        pltpu.VMEM_SHARED((ACC_ROWS, OUT_FEAT), jnp.float32),
        pltpu.VMEM((CHUNK,), jnp.int32),
        pltpu.VMEM((CHUNK,), jnp.int32),
        pltpu.VMEM((CHUNK,), jnp.float32),
        pltpu.VMEM((CHUNK, OUT_FEAT), jnp.float32),
        pltpu.VMEM((ZROWS, OUT_FEAT), jnp.float32),
        pltpu.SemaphoreType.DMA,
    ],
)(_sc_body)


def kernel(feat, edge_index, edge_weight, W):
    src = edge_index[1].astype(jnp.int32)
    dst = edge_index[0].astype(jnp.int32)
    ew = edge_weight.astype(jnp.float32)

    n_blocks = N_NODES // MM_BLK
    h = pl.pallas_call(
        _matmul_body,
        grid=(n_blocks,),
        in_specs=[
            pl.BlockSpec((MM_BLK, IN_FEAT), lambda i: (i, 0)),
            pl.BlockSpec((IN_FEAT, OUT_FEAT), lambda i: (0, 0)),
        ],
        out_specs=pl.BlockSpec((MM_BLK, OUT_FEAT), lambda i: (i, 0)),
        out_shape=jax.ShapeDtypeStruct((N_NODES, OUT_FEAT), jnp.float32),
    )(feat, W)

    partials = _sc_scatter(h, src, dst, ew)

    out = pl.pallas_call(
        _add_body,
        grid=(n_blocks,),
        in_specs=[
            pl.BlockSpec((MM_BLK, OUT_FEAT), lambda i: (i, 0)),
            pl.BlockSpec((MM_BLK, OUT_FEAT), lambda i: (i, 0)),
        ],
        out_specs=pl.BlockSpec((MM_BLK, OUT_FEAT), lambda i: (i, 0)),
        out_shape=jax.ShapeDtypeStruct((N_NODES, OUT_FEAT), jnp.float32),
    )(partials[0, :N_NODES], partials[1, :N_NODES])
    return out
